# baseline (speedup 1.0000x reference)
import functools

import jax
import jax.numpy as jnp
from jax import lax
from jax.experimental import pallas as pl
from jax.experimental.pallas import tpu as pltpu
from jax.experimental.pallas import tpu_sc as plsc

NUM_CELLS = 500000
N_QUAD = 16
N_VERTICES = 250000
N_EDGES = 750000

TC_BLOCK = 2000
TC_GRID = NUM_CELLS // TC_BLOCK

NC = 2
NS = 16
NW = NC * NS
ROW_W = 128
CHUNK_ROWS = 16
CHUNK_W = CHUNK_ROWS * ROW_W

SFLAT = 9 * NUM_CELLS
SROWS = 36864
SPAD = SROWS * ROW_W
ROWS_TILE = SROWS // NW
N_CHUNKS = ROWS_TILE // CHUNK_ROWS

VACC = 250112
ACC = 1751040
ACC_TILE = ACC // NS
CP = 4560
N_CP = ACC_TILE // CP

CB_ROWS = 13680
CB_BLK = 1368


def _integral_body(fx_ref, det_ref, faces_ref, f2e_ref, o_ref,
                   wa_ref, wb_ref, wf_ref, sv_ref, si_ref, face_ref):
    fx = fx_ref[...]
    det = det_ref[...]
    of = o_ref[...].astype(jnp.float32)
    o9 = jnp.concatenate(
        [jnp.ones((TC_BLOCK, 3), jnp.float32), of, of], axis=1)
    ya = jnp.dot(fx, wa_ref[...], preferred_element_type=jnp.float32)
    yb = jnp.dot(fx, wb_ref[...], preferred_element_type=jnp.float32)
    yf = jnp.dot(fx, wf_ref[...], preferred_element_type=jnp.float32)
    sv_ref[...] = (o9 * ya + (1.0 - o9) * yb) * det
    e2 = VACC + 2 * f2e_ref[...]
    si_ref[...] = jnp.concatenate([faces_ref[...], e2, e2 + 1], axis=1)
    face_ref[...] = yf * det


def _integral(f_x, det2, faces, f2e, orient, wa, wb, wf):
    row_spec = lambda w: pl.BlockSpec((TC_BLOCK, w), lambda i: (i, 0))
    full_spec = lambda a: pl.BlockSpec(a.shape, lambda i: (0, 0))
    return pl.pallas_call(
        _integral_body,
        grid=(TC_GRID,),
        in_specs=[row_spec(N_QUAD), row_spec(1), row_spec(3), row_spec(3),
                  row_spec(3), full_spec(wa), full_spec(wb), full_spec(wf)],
        out_specs=[row_spec(9), row_spec(9), row_spec(1)],
        out_shape=[
            jax.ShapeDtypeStruct((NUM_CELLS, 9), jnp.float32),
            jax.ShapeDtypeStruct((NUM_CELLS, 9), jnp.int32),
            jax.ShapeDtypeStruct((NUM_CELLS, 1), jnp.float32),
        ],
    )(f_x, det2, faces, f2e, orient, wa, wb, wf)


def _scatter_body(svals, sidx, hout, acc,
                  ib0, vb0, ib1, vb1, cp, ls0, ls1, ssem, osem):
    c = lax.axis_index("c")
    s = lax.axis_index("s")
    wid = c * NS + s

    zvec = jnp.zeros((16,), jnp.float32)

    def zfill(i, carry):
        cp[pl.ds(i * 16, 16)] = zvec
        return carry
    lax.fori_loop(0, CP // 16, zfill, 0)
    for k in range(N_CP):
        pltpu.async_copy(cp, acc.at[pl.ds(s * ACC_TILE + k * CP, CP)], osem)
    for k in range(N_CP):
        pltpu.make_async_copy(
            cp, acc.at[pl.ds(s * ACC_TILE + k * CP, CP)], osem).wait()

    plsc.subcore_barrier()

    def load_start(t, ib, vb, sem):
        row0 = wid * ROWS_TILE + t * CHUNK_ROWS
        pltpu.async_copy(sidx.at[pl.ds(row0, CHUNK_ROWS)], ib, sem)
        pltpu.async_copy(svals.at[pl.ds(row0 * ROW_W, CHUNK_W)], vb, sem)

    def load_wait(ib, vb, sem):
        pltpu.make_async_copy(sidx.at[pl.ds(0, CHUNK_ROWS)], ib, sem).wait()
        pltpu.make_async_copy(svals.at[pl.ds(0, CHUNK_W)], vb, sem).wait()

    def scatter(ib, vb):
        for j in range(CHUNK_ROWS):
            pltpu.async_copy(vb.at[pl.ds(j * ROW_W, ROW_W)],
                             acc.at[ib.at[j]], ssem, add=True)
        for j in range(CHUNK_ROWS):
            pltpu.make_async_copy(vb.at[pl.ds(j * ROW_W, ROW_W)],
                                  acc.at[ib.at[j]], ssem).wait()

    load_start(0, ib0, vb0, ls0)
    load_start(1, ib1, vb1, ls1)

    def pipe(p, carry):
        t0 = 2 * p
        load_wait(ib0, vb0, ls0)
        scatter(ib0, vb0)

        @pl.when(t0 + 2 < N_CHUNKS)
        def _():
            load_start(t0 + 2, ib0, vb0, ls0)
        load_wait(ib1, vb1, ls1)
        scatter(ib1, vb1)

        @pl.when(t0 + 3 < N_CHUNKS)
        def _():
            load_start(t0 + 3, ib1, vb1, ls1)
        return carry
    lax.fori_loop(0, N_CHUNKS // 2, pipe, 0)

    plsc.subcore_barrier()

    for k in range(N_CP):
        off = s * ACC_TILE + k * CP
        pltpu.sync_copy(acc.at[pl.ds(off, CP)], cp)
        pltpu.sync_copy(cp, hout.at[pl.ds(c * ACC + off, CP)])


_scatter = functools.partial(
    pl.kernel,
    out_type=jax.ShapeDtypeStruct((NC * ACC,), jnp.float32),
    mesh=plsc.VectorSubcoreMesh(core_axis_name="c", subcore_axis_name="s"),
    compiler_params=pltpu.CompilerParams(use_tc_tiling_on_sc=False),
    scratch_types=[
        pltpu.VMEM_SHARED((ACC,), jnp.float32),
        pltpu.VMEM((CHUNK_ROWS, ROW_W), jnp.int32),
        pltpu.VMEM((CHUNK_W,), jnp.float32),
        pltpu.VMEM((CHUNK_ROWS, ROW_W), jnp.int32),
        pltpu.VMEM((CHUNK_W,), jnp.float32),
        pltpu.VMEM((CP,), jnp.float32),
        pltpu.SemaphoreType.DMA,
        pltpu.SemaphoreType.DMA,
        pltpu.SemaphoreType.DMA,
        pltpu.SemaphoreType.DMA,
    ],
)(_scatter_body)


def _combine_body(in_ref, out_ref):
    out_ref[...] = in_ref[0] + in_ref[1]


def _combine(hout2):
    return pl.pallas_call(
        _combine_body,
        grid=(CB_ROWS // CB_BLK,),
        in_specs=[pl.BlockSpec((2, CB_BLK, 128), lambda i: (0, i, 0))],
        out_specs=pl.BlockSpec((CB_BLK, 128), lambda i: (i, 0)),
        out_shape=jax.ShapeDtypeStruct((CB_ROWS, 128), jnp.float32),
    )(hout2)


def kernel(f_x, v_x, quad_weights, det_A, faces, faces_to_edges,
           faces_to_edge_orientation):
    w = v_x * quad_weights[None, :]
    wa = w[jnp.array([0, 1, 2, 3, 5, 7, 4, 6, 8])].T
    wb = w[jnp.array([0, 1, 2, 4, 6, 8, 3, 5, 7])].T
    wf = w[9:10].T
    det2 = det_A[:, None]

    sv, si, face_dofs = _integral(
        f_x, det2, faces, faces_to_edges, faces_to_edge_orientation,
        wa, wb, wf)

    svals = jnp.pad(sv.reshape(SFLAT), (0, SPAD - SFLAT))
    sidx = jnp.pad(si.reshape(SFLAT), (0, SPAD - SFLAT)).reshape(
        SROWS, ROW_W)

    hout = _scatter(svals, sidx)
    fin = _combine(hout.reshape(NC, CB_ROWS, 128)).reshape(ACC)

    vertex_dofs = fin[:N_VERTICES]
    edge_dofs = fin[VACC:VACC + 2 * N_EDGES].reshape(N_EDGES, 2)
    return (vertex_dofs, edge_dofs, face_dofs)
